# Initial kernel scaffold; baseline (speedup 1.0000x reference)
import jax
import jax.numpy as jnp
from jax.experimental import pallas as pl


def kernel(feat_reticle, feat_dram, feat_link, edge_reticle, edge_dram, edge_link, W_reticle, b_reticle, W_dram, b_dram, W_link, b_link, W_task, b_task):
    raise NotImplementedError("write your pallas kernel here")


# speedup vs baseline: 2.6733x; 2.6733x over previous
import functools
import jax
import jax.numpy as jnp
from jax import lax
from jax.experimental import pallas as pl
from jax.experimental.pallas import tpu as pltpu
from jax.experimental.pallas import tpu_sc as plsc

N = 100000
E = 1600000
H = 64
NP = 100352
NB = 98
NBP = 112
NC, NS, NW = 2, 16, 32
EW = E // NW
CH = 2000
NGC = CH // 16
ROWS = E // 128
ROWSP = 12544
RPT = 392
EB = E + NW * NBP * 16 + 256
NEG = -3.0e38
POS = 3.0e38

_mesh = plsc.VectorSubcoreMesh(core_axis_name="c", subcore_axis_name="s")
_CP = pltpu.CompilerParams(needs_layout_passes=False)


def _wid():
    return lax.axis_index("s") * NC + lax.axis_index("c")


def _iota():
    return lax.iota(jnp.int32, 16)


def _sums_body(d_r, d_d, d_l, f_r, f_d, f_l, out, acc, ib, vb):
    wid = _wid()
    zero = jnp.zeros((16,), jnp.float32)
    ones = jnp.ones((16,), jnp.float32)
    dsts = [d_r, d_d, d_l, d_l]
    vals = [f_r, f_d, f_l, None]
    for t in range(4):
        def z(i, c):
            acc[pl.ds(i * 16, 16)] = zero
            return c
        lax.fori_loop(0, N // 16, z, jnp.int32(0))

        def chunk(k, c, t=t):
            base = wid * EW + k * CH
            pltpu.sync_copy(dsts[t].at[pl.ds(base, CH)], ib)
            if vals[t] is not None:
                pltpu.sync_copy(vals[t].at[pl.ds(base, CH)], vb)

            def grp(g, c2, t=t):
                idx = ib[pl.ds(g * 16, 16)]
                v = vb[pl.ds(g * 16, 16)] if vals[t] is not None else ones
                plsc.addupdate_scatter(acc, [idx], v)
                return c2
            return lax.fori_loop(0, NGC, grp, c)
        lax.fori_loop(0, 25, chunk, jnp.int32(0))
        pltpu.sync_copy(acc, out.at[pl.ds((t * NW + wid) * NP, N)])


_k_sums = pl.kernel(
    _sums_body, compiler_params=_CP, mesh=_mesh,
    out_type=jax.ShapeDtypeStruct((4 * NW * NP,), jnp.float32),
    scratch_types=[pltpu.VMEM((N,), jnp.float32),
                   pltpu.VMEM((CH,), jnp.int32),
                   pltpu.VMEM((CH,), jnp.float32)],
)


def _rsum_body(p_ref, o_ref):
    o_ref[...] = jnp.sum(p_ref[...], axis=1)


def _k_rsum(p1):
    blk = 2048
    return pl.pallas_call(
        _rsum_body,
        grid=(NP // blk,),
        in_specs=[pl.BlockSpec((4, NW, blk), lambda i: (0, 0, i))],
        out_specs=pl.BlockSpec((4, blk), lambda i: (0, i)),
        out_shape=jax.ShapeDtypeStruct((4, NP), jnp.float32),
    )(p1)


def _minmax_body(e0_r, e1_r, e0_d, e1_d, e0_l, e1_l, s_r, s_d, s_l,
                 pmax, pmin, gs, acc, e0b, e1b, gb, sem):
    wid = _wid()
    start = wid * RPT
    end = jnp.minimum(start + RPT, ROWS)
    nrows = jnp.maximum(end - start, 0)
    nch = (nrows + 7) // 8
    iota = _iota()
    e0s = [e0_r, e0_d, e0_l]
    e1s = [e1_r, e1_d, e1_l]
    tabs = [s_r, s_d, s_l]

    for rel in range(3):
        for ismax in (True, False):
            init = jnp.full((16,), NEG if ismax else POS, jnp.float32)

            def z(i, c):
                acc[pl.ds(i * 16, 16)] = init
                return c
            lax.fori_loop(0, N // 16, z, jnp.int32(0))

            def chunk(k, c, rel=rel, ismax=ismax):
                r0 = start + k * 8
                nr = jnp.minimum(8, nrows - k * 8)
                pltpu.sync_copy(e0s[rel].at[pl.ds(r0, 8)], e0b)
                if ismax:
                    pltpu.sync_copy(e1s[rel].at[pl.ds(r0, 8)], e1b)

                    def fire(j, c2, rel=rel):
                        pltpu.async_copy(tabs[rel].at[e1b.at[j]],
                                         gb.at[j], sem)
                        return c2
                    lax.fori_loop(0, nr, fire, jnp.int32(0))

                    def wt(j, c2, rel=rel):
                        pltpu.make_async_copy(tabs[rel].at[e1b.at[j]],
                                              gb.at[j], sem).wait()
                        return c2
                    lax.fori_loop(0, nr, wt, jnp.int32(0))

                    pltpu.sync_copy(gb, gs.at[pl.ds(r0, 8)])
                else:
                    pltpu.sync_copy(gs.at[pl.ds(r0, 8)], gb)

                def rowloop(j, c2, ismax=ismax):
                    for cc in range(8):
                        k16 = e0b[j, pl.ds(cc * 16, 16)]
                        g16 = gb[j, pl.ds(cc * 16, 16)]
                        sk, sv = plsc.sort_key_val(k16, g16)
                        for d in (1, 2, 4, 8):
                            src = jnp.maximum(iota - d, 0)
                            pk = sk.at[src].get(mode="promise_in_bounds")
                            pv = sv.at[src].get(mode="promise_in_bounds")
                            cond = (iota >= d) & (pk == sk)
                            comb = jnp.maximum(sv, pv) if ismax else jnp.minimum(sv, pv)
                            sv = jnp.where(cond, comb, sv)
                        nk = sk.at[jnp.minimum(iota + 1, 15)].get(
                            mode="promise_in_bounds")
                        lastm = (iota == 15) | (nk != sk)
                        cur = plsc.load_gather(acc, [sk], mask=lastm)
                        upd = jnp.maximum(cur, sv) if ismax else jnp.minimum(cur, sv)
                        plsc.store_scatter(acc, [sk], upd, mask=lastm)
                    return c2
                lax.fori_loop(0, nr, rowloop, jnp.int32(0))
                return c
            lax.fori_loop(0, nch, chunk, jnp.int32(0))
            dst = pmax if ismax else pmin
            pltpu.sync_copy(acc, dst.at[pl.ds((rel * NW + wid) * NP, N)])


_k_minmax = pl.kernel(
    _minmax_body, compiler_params=_CP, mesh=_mesh,
    out_type=[jax.ShapeDtypeStruct((3 * NW * NP,), jnp.float32),
              jax.ShapeDtypeStruct((3 * NW * NP,), jnp.float32),
              jax.ShapeDtypeStruct((ROWSP, 128), jnp.float32)],
    scratch_types=[pltpu.VMEM((N,), jnp.float32),
                   pltpu.VMEM((8, 128), jnp.int32),
                   pltpu.VMEM((8, 128), jnp.int32),
                   pltpu.VMEM((8, 128), jnp.float32),
                   pltpu.SemaphoreType.DMA],
)


def _rmm_body(pmax_ref, pmin_ref, o_ref):
    mx = jnp.max(pmax_ref[...], axis=1)
    mn = jnp.min(pmin_ref[...], axis=1)
    o_ref[...] = jnp.concatenate([mx, mn], axis=0)


def _k_rmm(pmax, pmin):
    blk = 2048
    return pl.pallas_call(
        _rmm_body,
        grid=(NP // blk,),
        in_specs=[pl.BlockSpec((3, NW, blk), lambda i: (0, 0, i)),
                  pl.BlockSpec((3, NW, blk), lambda i: (0, 0, i))],
        out_specs=pl.BlockSpec((6, blk), lambda i: (0, i)),
        out_shape=jax.ShapeDtypeStruct((6, NP), jnp.float32),
    )(pmax, pmin)


def _dense_body(mm_ref, wc_ref, bc_ref, wt_ref, bt_ref, o_ref):
    mm = mm_ref[...]
    hs = []
    for r in range(3):
        smax = mm[:, r:r + 1]
        smin = mm[:, 3 + r:4 + r]
        w = wc_ref[r:r + 1, :]
        b = bc_ref[r:r + 1, :]
        has = smax > -1.0e37
        x = jnp.where(w >= 0, smax, smin)
        h = jnp.where(has, jnp.tanh(x * w + b), 0.0)
        hs.append(h)
    hcat = jnp.concatenate(hs, axis=1)
    ht = jnp.tanh(jnp.dot(hcat, wt_ref[...],
                          preferred_element_type=jnp.float32) + bt_ref[...])
    o_ref[:, 0:64] = ht
    o_ref[:, 64:128] = jnp.zeros_like(ht)


def _k_dense(mm, wc, bc, wt, bt):
    blk = 512
    return pl.pallas_call(
        _dense_body,
        grid=(NP // blk,),
        in_specs=[pl.BlockSpec((blk, 128), lambda i: (i, 0)),
                  pl.BlockSpec((3, 64), lambda i: (0, 0)),
                  pl.BlockSpec((3, 64), lambda i: (0, 0)),
                  pl.BlockSpec((192, 64), lambda i: (0, 0)),
                  pl.BlockSpec((1, 64), lambda i: (0, 0))],
        out_specs=pl.BlockSpec((blk, 128), lambda i: (i, 0)),
        out_shape=jax.ShapeDtypeStruct((NP, 128), jnp.float32),
    )(mm, wc, bc, wt, bt)


def _count_body(d_l, out, cntv, ib):
    wid = _wid()
    zero = jnp.zeros((16,), jnp.int32)
    ones = jnp.ones((16,), jnp.int32)
    for i in range(NBP // 16):
        cntv[pl.ds(i * 16, 16)] = zero

    def chunk(k, c):
        base = wid * EW + k * CH
        pltpu.sync_copy(d_l.at[pl.ds(base, CH)], ib)

        def grp(g, c2):
            d = ib[pl.ds(g * 16, 16)]
            b = lax.shift_right_logical(d, 10)
            plsc.addupdate_scatter(cntv, [b], ones)
            return c2
        return lax.fori_loop(0, NGC, grp, c)
    lax.fori_loop(0, 25, chunk, jnp.int32(0))
    pltpu.sync_copy(cntv, out.at[pl.ds(wid * NBP, NBP)])


_k_count = pl.kernel(
    _count_body, compiler_params=_CP, mesh=_mesh,
    out_type=jax.ShapeDtypeStruct((NW * NBP,), jnp.int32),
    scratch_types=[pltpu.VMEM((NBP,), jnp.int32),
                   pltpu.VMEM((CH,), jnp.int32)],
)


def _prefix_body(cntt_ref, cell_ref, bsbc_ref):
    cntt = cntt_ref[...]
    c16 = jnp.bitwise_and(cntt + 15, jnp.int32(~15)).astype(jnp.float32)
    iw = lax.broadcasted_iota(jnp.int32, (NW, NW), 0)
    jw = lax.broadcasted_iota(jnp.int32, (NW, NW), 1)
    t_incl = (iw <= jw).astype(jnp.float32)
    incl = jnp.dot(c16, t_incl, preferred_element_type=jnp.float32,
                   precision=lax.Precision.HIGHEST)
    rowtot = incl[:, NW - 1:NW]
    ib = lax.broadcasted_iota(jnp.int32, (NBP, NBP), 0)
    jb = lax.broadcasted_iota(jnp.int32, (NBP, NBP), 1)
    t_ex = (jb < ib).astype(jnp.float32)
    rowoff = jnp.dot(t_ex, rowtot, preferred_element_type=jnp.float32,
                     precision=lax.Precision.HIGHEST)
    ex = incl - c16 + rowoff
    cell_ref[...] = ex.astype(jnp.int32)
    bsbc_ref[:, 0:1] = ex[:, 0:1].astype(jnp.int32)
    bsbc_ref[:, 1:2] = rowtot.astype(jnp.int32)


def _k_prefix(cntt):
    return pl.pallas_call(
        _prefix_body,
        grid=(1,),
        in_specs=[pl.BlockSpec((NBP, NW), lambda i: (0, 0))],
        out_specs=[pl.BlockSpec((NBP, NW), lambda i: (0, 0)),
                   pl.BlockSpec((NBP, 2), lambda i: (0, 0))],
        out_shape=[jax.ShapeDtypeStruct((NBP, NW), jnp.int32),
                   jax.ShapeDtypeStruct((NBP, 2), jnp.int32)],
    )(cntt)


def _binsc_body(s_l, d_l, cellt, bsrc, bdst, fillv, sb, db, stg_s, stg_d, sem):
    wid = _wid()
    iota = _iota()
    pltpu.sync_copy(cellt.at[pl.ds(wid * NBP, NBP)], fillv)

    def chunk(k, c):
        base = wid * EW + k * CH
        pltpu.sync_copy(s_l.at[pl.ds(base, CH)], sb)
        pltpu.sync_copy(d_l.at[pl.ds(base, CH)], db)

        def grp(g, c2):
            e0 = sb[pl.ds(g * 16, 16)]
            e1 = db[pl.ds(g * 16, 16)]
            b = lax.shift_right_logical(e1, 10)
            dl = jnp.bitwise_and(e1, 1023)
            sk, sperm = plsc.sort_key_val(b, iota)
            se0 = e0.at[sperm].get(mode="promise_in_bounds")
            sdl = dl.at[sperm].get(mode="promise_in_bounds")
            seg = iota
            for d in (1, 2, 4, 8):
                src = jnp.maximum(iota - d, 0)
                pk = sk.at[src].get(mode="promise_in_bounds")
                ps = seg.at[src].get(mode="promise_in_bounds")
                cond = (iota >= d) & (pk == sk)
                seg = jnp.where(cond, ps, seg)
            rank = iota - seg
            fills = plsc.load_gather(fillv, [sk])
            pos = fills + rank
            nk = sk.at[jnp.minimum(iota + 1, 15)].get(mode="promise_in_bounds")
            lastm = (iota == 15) | (nk != sk)
            plsc.store_scatter(fillv, [sk], pos + 1, mask=lastm)
            slot = jnp.bitwise_and(g, 31)
            stg_s[pl.ds(slot * 16, 16)] = se0
            stg_d[pl.ds(slot * 16, 16)] = sdl
            pltpu.async_copy(stg_s.at[pl.ds(slot * 16, 16)], bsrc.at[pos], sem)
            pltpu.async_copy(stg_d.at[pl.ds(slot * 16, 16)], bdst.at[pos], sem)

            @pl.when(g >= 16)
            def _():
                for _i in range(2):
                    pltpu.make_async_copy(s_l.at[pl.ds(0, 16)],
                                          stg_s.at[pl.ds(992, 16)], sem).wait()
            return c2
        lax.fori_loop(0, NGC, grp, c)
        for _i in range(32):
            pltpu.make_async_copy(s_l.at[pl.ds(0, 16)],
                                  stg_s.at[pl.ds(992, 16)], sem).wait()
        return c
    lax.fori_loop(0, 25, chunk, jnp.int32(0))

    stg_s[pl.ds(0, 16)] = jnp.zeros((16,), jnp.int32)
    stg_d[pl.ds(0, 16)] = jnp.full((16,), 1024, jnp.int32)
    for b in range(NB):
        bidx = jnp.full((16,), b, jnp.int32)
        fl = plsc.load_gather(fillv, [bidx])
        fl0 = fl[0]
        pad = jnp.bitwise_and(16 - jnp.bitwise_and(fl0, 15), 15)
        pos = jnp.where(iota < pad, fl0 + iota, (EB - 16) + iota)
        pltpu.async_copy(stg_s.at[pl.ds(0, 16)], bsrc.at[pos], sem).wait()
        pltpu.async_copy(stg_d.at[pl.ds(0, 16)], bdst.at[pos], sem).wait()

    @pl.when(wid == NW - 1)
    def _():
        def zt(i, c):
            stg_s[pl.ds(i * 16, 16)] = jnp.zeros((16,), jnp.int32)
            return c
        lax.fori_loop(0, 16, zt, jnp.int32(0))
        f97 = plsc.load_gather(fillv, [jnp.full((16,), NB - 1, jnp.int32)])[0]
        end16 = pl.multiple_of(jnp.bitwise_and(f97 + 15, -16), 16)
        pltpu.sync_copy(stg_s.at[pl.ds(0, 256)], bsrc.at[pl.ds(end16, 256)])
        pltpu.sync_copy(stg_s.at[pl.ds(0, 256)], bdst.at[pl.ds(end16, 256)])


_k_binsc = pl.kernel(
    _binsc_body, compiler_params=_CP, mesh=_mesh,
    out_type=[jax.ShapeDtypeStruct((EB,), jnp.int32),
              jax.ShapeDtypeStruct((EB,), jnp.int32)],
    scratch_types=[pltpu.VMEM((NBP,), jnp.int32),
                   pltpu.VMEM((CH,), jnp.int32),
                   pltpu.VMEM((CH,), jnp.int32),
                   pltpu.VMEM((1024,), jnp.int32),
                   pltpu.VMEM((1024,), jnp.int32),
                   pltpu.SemaphoreType.DMA],
)


def _s3_body(bsrc, bdst, bsbc, hpad, out, acc2, srcb, dstb, rows_v, bsv, sem):
    wid = _wid()
    pltpu.sync_copy(bsbc, bsv)
    zero = jnp.zeros((16,), jnp.float32)
    for r in range(4):
        b = wid + 32 * r
        if r == 3:
            do = b < NB
        else:
            do = b < NB

        @pl.when(do)
        def _(b=b):
            def z(i, c):
                acc2[pl.ds(i * 16, 16)] = zero
                return c
            lax.fori_loop(0, (1026 * 64) // 16, z, jnp.int32(0))
            bvec = jnp.full((16,), b, jnp.int32)
            bs = plsc.load_gather(bsv, [bvec])[0]
            bc = plsc.load_gather(bsv, [bvec + NBP])[0]
            nch = lax.shift_right_logical(bc + 63, 6)

            def chunk(k, c):
                off = pl.multiple_of(bs + k * 64, 16)
                pltpu.sync_copy(bsrc.at[pl.ds(off, 64)], srcb)
                pltpu.sync_copy(bdst.at[pl.ds(off, 64)], dstb)
                pltpu.async_copy(hpad.at[srcb], rows_v, sem).wait()
                ng = jnp.minimum(4, lax.shift_right_logical(bc - k * 64, 4))

                def grp(g, c2):
                    dl = dstb[pl.ds(g * 16, 16)]
                    base = dl * 64
                    for j in range(16):
                        loc = base[j]
                        for q in range(4):
                            v = rows_v[g * 16 + j, pl.ds(q * 16, 16)]
                            plsc.addupdate(acc2.at[pl.ds(loc + q * 16, 16)], v)
                    return c2
                lax.fori_loop(0, ng, grp, jnp.int32(0))
                return c
            lax.fori_loop(0, nch, chunk, jnp.int32(0))
            pltpu.sync_copy(acc2.at[pl.ds(0, 1024 * 64)],
                            out.at[pl.ds(b * 1024 * 64, 1024 * 64)])


_k_s3 = pl.kernel(
    _s3_body, compiler_params=_CP, mesh=_mesh,
    out_type=jax.ShapeDtypeStruct((NP * 64,), jnp.float32),
    scratch_types=[pltpu.VMEM((1026 * 64,), jnp.float32),
                   pltpu.VMEM((64,), jnp.int32),
                   pltpu.VMEM((64,), jnp.int32),
                   pltpu.VMEM((64, 128), jnp.float32),
                   pltpu.VMEM((2 * NBP,), jnp.int32),
                   pltpu.SemaphoreType.DMA],
)


def _final_body(s3_ref, cnt_ref, o_ref):
    cnt = cnt_ref[...]
    rcp = 1.0 / jnp.maximum(cnt, 1.0)
    o_ref[...] = s3_ref[...] * rcp[0][:, None]


def _k_final(s3, cnt):
    blk = 2048
    return pl.pallas_call(
        _final_body,
        grid=(NP // blk,),
        in_specs=[pl.BlockSpec((blk, 64), lambda i: (i, 0)),
                  pl.BlockSpec((1, blk), lambda i: (0, i))],
        out_specs=pl.BlockSpec((blk, 64), lambda i: (i, 0)),
        out_shape=jax.ShapeDtypeStruct((NP, 64), jnp.float32),
    )(s3, cnt)


def _rows(a):
    return jnp.pad(a, (0, ROWSP * 128 - E)).reshape(ROWSP, 128)


def kernel(feat_reticle, feat_dram, feat_link,
           edge_reticle, edge_dram, edge_link,
           W_reticle, b_reticle, W_dram, b_dram, W_link, b_link,
           W_task, b_task):
    f_r = feat_reticle.reshape(E)
    f_d = feat_dram.reshape(E)
    f_l = feat_link.reshape(E)
    s_r, d_r = edge_reticle[0], edge_reticle[1]
    s_d, d_d = edge_dram[0], edge_dram[1]
    s_l, d_l = edge_link[0], edge_link[1]

    p1 = _k_sums(d_r, d_d, d_l, f_r, f_d, f_l).reshape(4, NW, NP)
    ssum = _k_rsum(p1)

    pmax, pmin, _gs = _k_minmax(
        _rows(s_r), _rows(d_r), _rows(s_d), _rows(d_d), _rows(s_l), _rows(d_l),
        ssum[0], ssum[1], ssum[2])
    mm = _k_rmm(pmax.reshape(3, NW, NP), pmin.reshape(3, NW, NP))

    wc = jnp.stack([W_reticle[0], W_dram[0], W_link[0]])
    bc = jnp.stack([b_reticle, b_dram, b_link])
    mmp = jnp.pad(mm.T, ((0, 0), (0, 122)))
    hpad = _k_dense(mmp, wc, bc, W_task, b_task.reshape(1, H))

    cnt = _k_count(d_l).reshape(NW, NBP)
    cell, bsbc = _k_prefix(cnt.T)
    bsrc, bdst = _k_binsc(s_l, d_l, cell.T.reshape(-1))
    bsbc_flat = jnp.concatenate([bsbc[:, 0], bsbc[:, 1]])
    s3 = _k_s3(bsrc, bdst, bsbc_flat, hpad).reshape(NP, 64)
    hl = _k_final(s3, ssum[3:4])

    return (hpad[:N, :H], hl[:N])
